# baseline (speedup 1.0000x reference)
import functools

import jax
import jax.numpy as jnp
from jax import lax
from jax.experimental import pallas as pl
from jax.experimental.pallas import tpu as pltpu
from jax.experimental.pallas import tpu_sc as plsc

_N = 10000
_E = 160000
_L = 3
_D = 128
_H = 2
_ETYPES = (1, 3)

_EB = 256
_NEB = _E // _EB
_EPAD = _E + 2 * _EB

_C = 64
_G = _C // 16
_Q0 = 2504
_Q1 = 2496
_DUMP = 2520
_RPT = 160
_NROWS = 16 * _RPT
_OB = 40


def _tc_node_body(feat_ref, w1_ref, dst_ref, a1n_ref, split_ref, dstp_ref):
  f = feat_ref[...]
  s0 = jnp.sum(f * w1_ref[0:1, :], axis=1, keepdims=True)
  s1 = jnp.sum(f * w1_ref[1:2, :], axis=1, keepdims=True)
  a1n_ref[...] = jnp.concatenate([s0, s1], axis=1)
  d = dst_ref[...]
  lane = lax.broadcasted_iota(jnp.int32, (1, 16), 1)
  cnts = jnp.zeros((1, 16), jnp.int32)
  for k, thr in enumerate((_Q0, _Q0 + _Q1, _Q0 + _Q1 + _Q0)):
    cnt = jnp.sum((d < thr).astype(jnp.int32))
    cnts = jnp.where(lane == k, cnt, cnts)
  split_ref[...] = cnts
  dstp_ref[0:_NEB, :] = d
  dstp_ref[_NEB:, :] = jnp.zeros((2, _EB), jnp.int32)


def _tc_edge_body(ed0_ref, ed1_ref, ed2_ref, w_ref, a2w_ref, mp_ref, a2_ref):
  mp = jnp.dot(ed0_ref[0], w_ref[0], preferred_element_type=jnp.float32)
  mp += jnp.dot(ed1_ref[0], w_ref[1], preferred_element_type=jnp.float32)
  mp += jnp.dot(ed2_ref[0], w_ref[2], preferred_element_type=jnp.float32)
  mp_ref[...] = mp
  a2_ref[...] = jax.lax.dot_general(
      mp, a2w_ref[...], (((1,), (1,)), ((), ())),
      preferred_element_type=jnp.float32)


def _sc_body(mp_hbm, a2_hbm, dst_hbm, a1_hbm, split_hbm, out_hbm,
             a1_v, split_v, dst_v, a2_v, w0_v, w1_v, idx_v,
             vals0_v, vals1_v, den_v, s0_sh, s1_sh, d_sh):
  c = lax.axis_index("c")
  s = lax.axis_index("s")

  pltpu.sync_copy(split_hbm, split_v)
  pltpu.sync_copy(a1_hbm, a1_v)

  sp = split_v[pl.ds(0, 16)]
  sp0 = sp[0]
  sp1 = sp[1]
  sp2 = sp[2]

  iota = lax.iota(jnp.int32, 16)
  oh0 = (iota == 0).astype(jnp.float32)
  oh1 = (iota == 1).astype(jnp.float32)

  def zden(r, carry):
    for k in range(_D // 16):
      den_v[r, pl.ds(16 * k, 16)] = jnp.zeros((16,), jnp.float32)
    return carry
  lax.fori_loop(0, _C, zden, 0)

  for p in range(2):
    if p == 0:
      estart = jnp.where(c == 0, 0, sp1)
      eend = jnp.where(c == 0, sp0, sp2)
      node_base = c * (_Q0 + _Q1)
      qsize = _Q0
    else:
      estart = jnp.where(c == 0, sp0, sp2)
      eend = jnp.where(c == 0, sp1, _E)
      node_base = _Q0 + c * (_Q0 + _Q1)
      qsize = _Q1

    def zrow(r, carry):
      for k in range(_D // 16):
        vals0_v[r, pl.ds(16 * k, 16)] = jnp.zeros((16,), jnp.float32)
      return carry
    lax.fori_loop(0, 16, zrow, 0)
    row0 = pl.multiple_of(s * _RPT, 8)
    for j in range(_RPT // 16):
      pltpu.sync_copy(vals0_v.at[pl.ds(0, 16), :],
                      s0_sh.at[pl.ds(row0 + j * 16, 16), :])
      pltpu.sync_copy(vals0_v.at[pl.ds(0, 16), :],
                      s1_sh.at[pl.ds(row0 + j * 16, 16), :])
      pltpu.sync_copy(vals0_v.at[pl.ds(0, 16), :],
                      d_sh.at[pl.ds(row0 + j * 16, 16), :])
    plsc.subcore_barrier()

    astart = (estart // 8) * 8
    cnt = eend - astart
    per = ((cnt + 127) // 128) * 8
    tstart = astart + s * per
    tend = jnp.minimum(tstart + per, eend)
    nch = (jnp.maximum(tend - tstart, 0) + (_C - 1)) // _C

    def chunk_body(j, carry):
      base = pl.multiple_of(tstart + j * _C, 8)
      pltpu.sync_copy(dst_hbm.at[pl.ds(base, _C)], dst_v)
      pltpu.sync_copy(a2_hbm.at[pl.ds(2 * base, 2 * _C)], a2_v)
      pltpu.sync_copy(mp_hbm.at[pl.ds(base, _C), :], vals0_v)
      for g in range(_G):
        dstv = dst_v[pl.ds(g * 16, 16)]
        eidx = base + g * 16 + iota
        mask = (eidx >= estart) & (eidx < tend)
        a1h0 = plsc.load_gather(a1_v, [2 * dstv])
        a1h1 = plsc.load_gather(a1_v, [2 * dstv + 1])
        a2h0 = plsc.load_gather(a2_v, [32 * g + 2 * iota])
        a2h1 = plsc.load_gather(a2_v, [32 * g + 2 * iota + 1])
        x0 = a1h0 + a2h0
        x0 = jnp.where(x0 > 0, x0, 0.01 * x0)
        x1 = a1h1 + a2h1
        x1 = jnp.where(x1 > 0, x1, 0.01 * x1)
        w0 = jnp.where(mask, jnp.exp(x0), 0.0)
        w1 = jnp.where(mask, jnp.exp(x1), 0.0)
        w0_v[pl.ds(g * 16, 16)] = w0
        w1_v[pl.ds(g * 16, 16)] = w1
        idx_v[pl.ds(g * 16, 16)] = jnp.where(mask, dstv - node_base, _DUMP)

      def edge_group(g2, carry2):
        wv0 = w0_v[pl.ds(g2 * 16, 16)]
        wv1 = w1_v[pl.ds(g2 * 16, 16)]
        for l in range(16):
          e = g2 * 16 + l
          w0 = wv0[l]
          w1 = wv1[l]
          for k in range(_D // 16):
            m = vals0_v[e, pl.ds(16 * k, 16)]
            vals1_v[e, pl.ds(16 * k, 16)] = m * w1
            vals0_v[e, pl.ds(16 * k, 16)] = m * w0
          den_v[e, pl.ds(0, 16)] = oh0 * w0 + oh1 * w1
        return carry2
      lax.fori_loop(0, _G, edge_group, 0)

      pltpu.sync_copy(vals0_v, s0_sh.at[idx_v], add=True)
      pltpu.sync_copy(vals1_v, s1_sh.at[idx_v], add=True)
      pltpu.sync_copy(den_v, d_sh.at[idx_v], add=True)
      return carry
    lax.fori_loop(0, nch, chunk_body, 0)

    plsc.subcore_barrier()

    rstart = s * _RPT
    rend = jnp.minimum(rstart + _RPT, qsize)
    rcnt = jnp.maximum(rend - rstart, 0)
    nch2 = (rcnt + _OB - 1) // _OB

    def out_body(ch, carry):
      r0 = pl.multiple_of(
          jnp.minimum(rstart + ch * _OB, rend - _OB), 8)
      pltpu.sync_copy(s0_sh.at[pl.ds(r0, _OB), :], vals0_v.at[pl.ds(0, _OB), :])
      pltpu.sync_copy(s1_sh.at[pl.ds(r0, _OB), :], vals1_v.at[pl.ds(0, _OB), :])
      pltpu.sync_copy(d_sh.at[pl.ds(r0, _OB), :], den_v.at[pl.ds(0, _OB), :])

      def row_body(r, carry2):
        den = den_v[r, pl.ds(0, 16)]
        invv = 1.0 / jnp.maximum(den, 1e-20)
        inv0 = invv[0]
        inv1 = invv[1]
        for k in range(_D // 16):
          x = vals0_v[r, pl.ds(16 * k, 16)] * inv0
          vals0_v[r, pl.ds(16 * k, 16)] = jnp.where(x > 0, x, jnp.exp(x) - 1.0)
        for k in range(_D // 16):
          x = vals1_v[r, pl.ds(16 * k, 16)] * inv1
          vals1_v[r, pl.ds(16 * k, 16)] = jnp.where(x > 0, x, jnp.exp(x) - 1.0)
        return carry2
      lax.fori_loop(0, _OB, row_body, 0)

      obase = pl.multiple_of(node_base + r0, 8)
      pltpu.sync_copy(vals0_v.at[pl.ds(0, _OB), :],
                      out_hbm.at[pl.ds(obase, _OB), pl.ds(0, _D)])
      pltpu.sync_copy(vals1_v.at[pl.ds(0, _OB), :],
                      out_hbm.at[pl.ds(obase, _OB), pl.ds(_D, _D)])
      return carry
    lax.fori_loop(0, nch2, out_body, 0)

    if p == 0:
      lax.fori_loop(0, _OB, zden, 0)
      plsc.subcore_barrier()


_sc_kernel = functools.partial(
    pl.kernel,
    out_type=jax.ShapeDtypeStruct((_N, 2 * _D), jnp.float32),
    mesh=plsc.VectorSubcoreMesh(core_axis_name="c", subcore_axis_name="s"),
    compiler_params=pltpu.CompilerParams(needs_layout_passes=False),
    scratch_types=[
        pltpu.VMEM((2 * _N,), jnp.float32),
        pltpu.VMEM((16,), jnp.int32),
        pltpu.VMEM((_C,), jnp.int32),
        pltpu.VMEM((2 * _C,), jnp.float32),
        pltpu.VMEM((_C,), jnp.float32),
        pltpu.VMEM((_C,), jnp.float32),
        pltpu.VMEM((_C,), jnp.int32),
        pltpu.VMEM((_C, _D), jnp.float32),
        pltpu.VMEM((_C, _D), jnp.float32),
        pltpu.VMEM((_C, _D), jnp.float32),
        pltpu.VMEM_SHARED((_NROWS, _D), jnp.float32),
        pltpu.VMEM_SHARED((_NROWS, _D), jnp.float32),
        pltpu.VMEM_SHARED((_NROWS, _D), jnp.float32),
    ],
)(_sc_body)


def _build_rot_weight(r_vec):
  rv = r_vec / jnp.maximum(jnp.linalg.norm(r_vec, axis=2, keepdims=True), 1e-12)
  rv2 = jnp.stack([rv, rv], axis=1)
  rv2 = rv2.at[:, 1, :, 1].set(-rv2[:, 1, :, 1])
  rv2 = rv2.reshape(r_vec.shape[0] * 2, _D // 2, 2)
  final = jnp.zeros((_L, _D // 2, 2), jnp.float32)
  final = final.at[-1, :, 0].set(1.0)
  for i in range(_L - 2, -1, -1):
    re = final[i + 1, :, 0] * rv2[_ETYPES[i], :, 0] - final[i + 1, :, 1] * rv2[_ETYPES[i], :, 1]
    im = final[i + 1, :, 0] * rv2[_ETYPES[i], :, 1] + final[i + 1, :, 1] * rv2[_ETYPES[i], :, 0]
    final = final.at[i, :, 0].set(re)
    final = final.at[i, :, 1].set(im)
  cc = final[:, :, 0]
  ss = final[:, :, 1]
  m = jnp.stack([jnp.stack([cc, ss], -1), jnp.stack([-ss, cc], -1)], -2) / 3.0
  return jnp.einsum("pq,ipab->ipaqb", jnp.eye(_D // 2, dtype=jnp.float32),
                    m).reshape(_L * _D, _D)


def kernel(features, edata, dst_index, attn1_w, attn2, r_vec,
           fusion_w1, fusion_b1, fusion_w2):
  del fusion_w1, fusion_b1, fusion_w2
  w_rot = _build_rot_weight(r_vec).reshape(_L, _D, _D)
  edata_t = jnp.transpose(edata, (1, 0, 2))
  dst2d = dst_index.astype(jnp.int32).reshape(_NEB, _EB)

  a1n, splitv, dstp2d = pl.pallas_call(
      _tc_node_body,
      out_shape=[
          jax.ShapeDtypeStruct((_N, 2), jnp.float32),
          jax.ShapeDtypeStruct((1, 16), jnp.int32),
          jax.ShapeDtypeStruct((_EPAD // _EB, _EB), jnp.int32),
      ],
  )(features, attn1_w, dst2d)

  mp, a2 = pl.pallas_call(
      _tc_edge_body,
      grid=(_EPAD // _EB,),
      in_specs=[
          pl.BlockSpec((1, _EB, _D),
                       lambda i: (0, jnp.minimum(i, _NEB - 1), 0)),
          pl.BlockSpec((1, _EB, _D),
                       lambda i: (1, jnp.minimum(i, _NEB - 1), 0)),
          pl.BlockSpec((1, _EB, _D),
                       lambda i: (2, jnp.minimum(i, _NEB - 1), 0)),
          pl.BlockSpec((_L, _D, _D), lambda i: (0, 0, 0)),
          pl.BlockSpec((_H, _D), lambda i: (0, 0)),
      ],
      out_specs=[
          pl.BlockSpec((_EB, _D), lambda i: (i, 0)),
          pl.BlockSpec((_EB, 2), lambda i: (i, 0)),
      ],
      out_shape=[
          jax.ShapeDtypeStruct((_EPAD, _D), jnp.float32),
          jax.ShapeDtypeStruct((_EPAD, 2), jnp.float32),
      ],
  )(edata_t, edata_t, edata_t, w_rot, attn2[0])

  return _sc_kernel(mp, a2.reshape(-1), dstp2d.reshape(-1), a1n.reshape(-1),
                    splitv.reshape(-1))


# speedup vs baseline: 1.2025x; 1.2025x over previous
import functools

import jax
import jax.numpy as jnp
from jax import lax
from jax.experimental import pallas as pl
from jax.experimental.pallas import tpu as pltpu
from jax.experimental.pallas import tpu_sc as plsc

_N = 10000
_E = 160000
_L = 3
_D = 128
_H = 2
_ETYPES = (1, 3)

_EB = 256
_NEB = _E // _EB
_EPAD = _E + 2 * _EB

_C = 64
_G = _C // 16
_Q0 = 2504
_Q1 = 2496
_DUMP = 2520
_RPT = 160
_NROWS = 16 * _RPT
_OB = 40


def _tc_node_body(feat_ref, w1_ref, dst_ref, a1n_ref, split_ref, dstp_ref):
  f = feat_ref[...]
  s0 = jnp.sum(f * w1_ref[0:1, :], axis=1, keepdims=True)
  s1 = jnp.sum(f * w1_ref[1:2, :], axis=1, keepdims=True)
  a1n_ref[...] = jnp.concatenate([s0, s1], axis=1)
  d = dst_ref[...]
  lane = lax.broadcasted_iota(jnp.int32, (1, 16), 1)
  cnts = jnp.zeros((1, 16), jnp.int32)
  for k, thr in enumerate((_Q0, _Q0 + _Q1, _Q0 + _Q1 + _Q0)):
    cnt = jnp.sum((d < thr).astype(jnp.int32))
    cnts = jnp.where(lane == k, cnt, cnts)
  split_ref[...] = cnts
  dstp_ref[0:_NEB, :] = d
  dstp_ref[_NEB:, :] = jnp.zeros((2, _EB), jnp.int32)


def _tc_edge_body(ed0_ref, ed1_ref, ed2_ref, w_ref, a2w_ref, mp_ref, a2_ref):
  mp = jnp.dot(ed0_ref[0], w_ref[0], preferred_element_type=jnp.float32)
  mp += jnp.dot(ed1_ref[0], w_ref[1], preferred_element_type=jnp.float32)
  mp += jnp.dot(ed2_ref[0], w_ref[2], preferred_element_type=jnp.float32)
  mp_ref[...] = mp
  s0 = jnp.sum(mp * a2w_ref[0:1, :], axis=1, keepdims=True)
  s1 = jnp.sum(mp * a2w_ref[1:2, :], axis=1, keepdims=True)
  a2_ref[...] = jnp.concatenate([s0, s1], axis=1)


def _sc_body(mp_hbm, a2_hbm, dst_hbm, a1_hbm, split_hbm, out_hbm,
             a1_v, split_v, dst_v, a2_v, mp_v, w0_v, w1_v, idx_v,
             vals0_v, vals1_v, den_v, sem_in, sem_sc, s0_sh, s1_sh, d_sh):
  c = lax.axis_index("c")
  s = lax.axis_index("s")

  pltpu.sync_copy(split_hbm, split_v)
  pltpu.sync_copy(a1_hbm, a1_v)

  sp = split_v[pl.ds(0, 16)]
  sp0 = sp[0]
  sp1 = sp[1]
  sp2 = sp[2]

  iota = lax.iota(jnp.int32, 16)
  oh0 = (iota == 0).astype(jnp.float32)
  oh1 = (iota == 1).astype(jnp.float32)

  def zden(r, carry):
    for k in range(_D // 16):
      den_v[r, pl.ds(16 * k, 16)] = jnp.zeros((16,), jnp.float32)
    return carry
  lax.fori_loop(0, _C, zden, 0)

  for p in range(2):
    if p == 0:
      estart = jnp.where(c == 0, 0, sp1)
      eend = jnp.where(c == 0, sp0, sp2)
      node_base = c * (_Q0 + _Q1)
      qsize = _Q0
    else:
      estart = jnp.where(c == 0, sp0, sp2)
      eend = jnp.where(c == 0, sp1, _E)
      node_base = _Q0 + c * (_Q0 + _Q1)
      qsize = _Q1

    def zrow(r, carry):
      for k in range(_D // 16):
        vals0_v[r, pl.ds(16 * k, 16)] = jnp.zeros((16,), jnp.float32)
      return carry
    lax.fori_loop(0, 16, zrow, 0)
    row0 = pl.multiple_of(s * _RPT, 8)
    for j in range(_RPT // 16):
      pltpu.sync_copy(vals0_v.at[pl.ds(0, 16), :],
                      s0_sh.at[pl.ds(row0 + j * 16, 16), :])
      pltpu.sync_copy(vals0_v.at[pl.ds(0, 16), :],
                      s1_sh.at[pl.ds(row0 + j * 16, 16), :])
      pltpu.sync_copy(vals0_v.at[pl.ds(0, 16), :],
                      d_sh.at[pl.ds(row0 + j * 16, 16), :])
    plsc.subcore_barrier()

    astart = (estart // 8) * 8
    cnt = eend - astart
    per = ((cnt + 127) // 128) * 8
    tstart = astart + s * per
    tend = jnp.minimum(tstart + per, eend)
    nch = (jnp.maximum(tend - tstart, 0) + (_C - 1)) // _C

    def chunk_body(j, carry):
      base = pl.multiple_of(tstart + j * _C, 8)
      d_dst = pltpu.async_copy(dst_hbm.at[pl.ds(base, _C)], dst_v, sem_in)
      d_a2 = pltpu.async_copy(a2_hbm.at[pl.ds(2 * base, 2 * _C)], a2_v, sem_in)
      d_mp = pltpu.async_copy(mp_hbm.at[pl.ds(base, _C), :], mp_v, sem_in)

      @pl.when(j > 0)
      def _drain():
        pltpu.make_async_copy(vals0_v, s0_sh.at[idx_v], sem_sc).wait()
        pltpu.make_async_copy(vals1_v, s1_sh.at[idx_v], sem_sc).wait()
        pltpu.make_async_copy(den_v, d_sh.at[idx_v], sem_sc).wait()

      d_dst.wait()
      d_a2.wait()
      d_mp.wait()
      for g in range(_G):
        dstv = dst_v[pl.ds(g * 16, 16)]
        eidx = base + g * 16 + iota
        mask = (eidx >= estart) & (eidx < tend)
        a1h0 = plsc.load_gather(a1_v, [2 * dstv])
        a1h1 = plsc.load_gather(a1_v, [2 * dstv + 1])
        a2h0 = plsc.load_gather(a2_v, [32 * g + 2 * iota])
        a2h1 = plsc.load_gather(a2_v, [32 * g + 2 * iota + 1])
        x0 = a1h0 + a2h0
        x0 = jnp.where(x0 > 0, x0, 0.01 * x0)
        x1 = a1h1 + a2h1
        x1 = jnp.where(x1 > 0, x1, 0.01 * x1)
        w0 = jnp.where(mask, jnp.exp(x0), 0.0)
        w1 = jnp.where(mask, jnp.exp(x1), 0.0)
        w0_v[pl.ds(g * 16, 16)] = w0
        w1_v[pl.ds(g * 16, 16)] = w1
        idx_v[pl.ds(g * 16, 16)] = jnp.where(mask, dstv - node_base, _DUMP)

      def edge_group(g2, carry2):
        wv0 = w0_v[pl.ds(g2 * 16, 16)]
        wv1 = w1_v[pl.ds(g2 * 16, 16)]
        for l in range(16):
          e = g2 * 16 + l
          w0 = wv0[l]
          w1 = wv1[l]
          for k in range(_D // 16):
            m = mp_v[e, pl.ds(16 * k, 16)]
            vals0_v[e, pl.ds(16 * k, 16)] = m * w0
            vals1_v[e, pl.ds(16 * k, 16)] = m * w1
          den_v[e, pl.ds(0, 16)] = oh0 * w0 + oh1 * w1
        return carry2
      lax.fori_loop(0, _G, edge_group, 0)

      pltpu.async_copy(vals0_v, s0_sh.at[idx_v], sem_sc, add=True)
      pltpu.async_copy(vals1_v, s1_sh.at[idx_v], sem_sc, add=True)
      pltpu.async_copy(den_v, d_sh.at[idx_v], sem_sc, add=True)
      return carry
    lax.fori_loop(0, nch, chunk_body, 0)

    @pl.when(nch > 0)
    def _final_drain():
      pltpu.make_async_copy(vals0_v, s0_sh.at[idx_v], sem_sc).wait()
      pltpu.make_async_copy(vals1_v, s1_sh.at[idx_v], sem_sc).wait()
      pltpu.make_async_copy(den_v, d_sh.at[idx_v], sem_sc).wait()

    plsc.subcore_barrier()

    rstart = s * _RPT
    rend = jnp.minimum(rstart + _RPT, qsize)
    rcnt = jnp.maximum(rend - rstart, 0)
    nch2 = (rcnt + _OB - 1) // _OB

    def out_body(ch, carry):
      r0 = pl.multiple_of(
          jnp.minimum(rstart + ch * _OB, rend - _OB), 8)
      pltpu.sync_copy(s0_sh.at[pl.ds(r0, _OB), :], vals0_v.at[pl.ds(0, _OB), :])
      pltpu.sync_copy(s1_sh.at[pl.ds(r0, _OB), :], vals1_v.at[pl.ds(0, _OB), :])
      pltpu.sync_copy(d_sh.at[pl.ds(r0, _OB), :], den_v.at[pl.ds(0, _OB), :])

      def row_body(r, carry2):
        den = den_v[r, pl.ds(0, 16)]
        invv = 1.0 / jnp.maximum(den, 1e-20)
        inv0 = invv[0]
        inv1 = invv[1]
        for k in range(_D // 16):
          x = vals0_v[r, pl.ds(16 * k, 16)] * inv0
          vals0_v[r, pl.ds(16 * k, 16)] = jnp.where(x > 0, x, jnp.exp(x) - 1.0)
        for k in range(_D // 16):
          x = vals1_v[r, pl.ds(16 * k, 16)] * inv1
          vals1_v[r, pl.ds(16 * k, 16)] = jnp.where(x > 0, x, jnp.exp(x) - 1.0)
        return carry2
      lax.fori_loop(0, _OB, row_body, 0)

      obase = pl.multiple_of(node_base + r0, 8)
      pltpu.sync_copy(vals0_v.at[pl.ds(0, _OB), :],
                      out_hbm.at[pl.ds(obase, _OB), pl.ds(0, _D)])
      pltpu.sync_copy(vals1_v.at[pl.ds(0, _OB), :],
                      out_hbm.at[pl.ds(obase, _OB), pl.ds(_D, _D)])
      return carry
    lax.fori_loop(0, nch2, out_body, 0)

    if p == 0:
      lax.fori_loop(0, _OB, zden, 0)
      plsc.subcore_barrier()


_sc_kernel = functools.partial(
    pl.kernel,
    out_type=jax.ShapeDtypeStruct((_N, 2 * _D), jnp.float32),
    mesh=plsc.VectorSubcoreMesh(core_axis_name="c", subcore_axis_name="s"),
    compiler_params=pltpu.CompilerParams(needs_layout_passes=False),
    scratch_types=[
        pltpu.VMEM((2 * _N,), jnp.float32),
        pltpu.VMEM((16,), jnp.int32),
        pltpu.VMEM((_C,), jnp.int32),
        pltpu.VMEM((2 * _C,), jnp.float32),
        pltpu.VMEM((_C, _D), jnp.float32),
        pltpu.VMEM((_C,), jnp.float32),
        pltpu.VMEM((_C,), jnp.float32),
        pltpu.VMEM((_C,), jnp.int32),
        pltpu.VMEM((_C, _D), jnp.float32),
        pltpu.VMEM((_C, _D), jnp.float32),
        pltpu.VMEM((_C, _D), jnp.float32),
        pltpu.SemaphoreType.DMA,
        pltpu.SemaphoreType.DMA,
        pltpu.VMEM_SHARED((_NROWS, _D), jnp.float32),
        pltpu.VMEM_SHARED((_NROWS, _D), jnp.float32),
        pltpu.VMEM_SHARED((_NROWS, _D), jnp.float32),
    ],
)(_sc_body)


def _build_rot_weight(r_vec):
  rv = r_vec / jnp.maximum(jnp.linalg.norm(r_vec, axis=2, keepdims=True), 1e-12)
  rv2 = jnp.stack([rv, rv], axis=1)
  rv2 = rv2.at[:, 1, :, 1].set(-rv2[:, 1, :, 1])
  rv2 = rv2.reshape(r_vec.shape[0] * 2, _D // 2, 2)
  final = jnp.zeros((_L, _D // 2, 2), jnp.float32)
  final = final.at[-1, :, 0].set(1.0)
  for i in range(_L - 2, -1, -1):
    re = final[i + 1, :, 0] * rv2[_ETYPES[i], :, 0] - final[i + 1, :, 1] * rv2[_ETYPES[i], :, 1]
    im = final[i + 1, :, 0] * rv2[_ETYPES[i], :, 1] + final[i + 1, :, 1] * rv2[_ETYPES[i], :, 0]
    final = final.at[i, :, 0].set(re)
    final = final.at[i, :, 1].set(im)
  cc = final[:, :, 0]
  ss = final[:, :, 1]
  m = jnp.stack([jnp.stack([cc, ss], -1), jnp.stack([-ss, cc], -1)], -2) / 3.0
  return jnp.einsum("pq,ipab->ipaqb", jnp.eye(_D // 2, dtype=jnp.float32),
                    m).reshape(_L * _D, _D)


def kernel(features, edata, dst_index, attn1_w, attn2, r_vec,
           fusion_w1, fusion_b1, fusion_w2):
  del fusion_w1, fusion_b1, fusion_w2
  w_rot = _build_rot_weight(r_vec).reshape(_L, _D, _D)
  edata_t = jnp.transpose(edata, (1, 0, 2))
  dst2d = dst_index.astype(jnp.int32).reshape(_NEB, _EB)

  a1n, splitv, dstp2d = pl.pallas_call(
      _tc_node_body,
      out_shape=[
          jax.ShapeDtypeStruct((_N, 2), jnp.float32),
          jax.ShapeDtypeStruct((1, 16), jnp.int32),
          jax.ShapeDtypeStruct((_EPAD // _EB, _EB), jnp.int32),
      ],
  )(features, attn1_w, dst2d)

  mp, a2 = pl.pallas_call(
      _tc_edge_body,
      grid=(_EPAD // _EB,),
      in_specs=[
          pl.BlockSpec((1, _EB, _D),
                       lambda i: (0, jnp.minimum(i, _NEB - 1), 0)),
          pl.BlockSpec((1, _EB, _D),
                       lambda i: (1, jnp.minimum(i, _NEB - 1), 0)),
          pl.BlockSpec((1, _EB, _D),
                       lambda i: (2, jnp.minimum(i, _NEB - 1), 0)),
          pl.BlockSpec((_L, _D, _D), lambda i: (0, 0, 0)),
          pl.BlockSpec((_H, _D), lambda i: (0, 0)),
      ],
      out_specs=[
          pl.BlockSpec((_EB, _D), lambda i: (i, 0)),
          pl.BlockSpec((_EB, 2), lambda i: (i, 0)),
      ],
      out_shape=[
          jax.ShapeDtypeStruct((_EPAD, _D), jnp.float32),
          jax.ShapeDtypeStruct((_EPAD, 2), jnp.float32),
      ],
  )(edata_t, edata_t, edata_t, w_rot, attn2[0])

  return _sc_kernel(mp, a2.reshape(-1), dstp2d.reshape(-1), a1n.reshape(-1),
                    splitv.reshape(-1))


# speedup vs baseline: 1.3145x; 1.0932x over previous
import functools

import jax
import jax.numpy as jnp
from jax import lax
from jax.experimental import pallas as pl
from jax.experimental.pallas import tpu as pltpu
from jax.experimental.pallas import tpu_sc as plsc

_N = 10000
_E = 160000
_L = 3
_D = 128
_H = 2
_ETYPES = (1, 3)

_EB = 256
_NEB = _E // _EB
_NEBP = 634
_EPAD = _NEBP * _EB

_C = 64
_G = _C // 16
_Q0 = 2504
_Q1 = 2496
_DUMP = 2520
_RPT = 160
_NROWS = 16 * _RPT
_OB = 40


def _tc_node_body(feat_ref, w1_ref, dst_ref, a1n_ref, split_ref, dstp_ref):
  f = feat_ref[...]
  s0 = jnp.sum(f * w1_ref[0:1, :], axis=1, keepdims=True)
  s1 = jnp.sum(f * w1_ref[1:2, :], axis=1, keepdims=True)
  a1n_ref[...] = jnp.concatenate([s0, s1], axis=1)
  d = dst_ref[...]
  lane = lax.broadcasted_iota(jnp.int32, (1, 16), 1)
  cnts = jnp.zeros((1, 16), jnp.int32)
  for k, thr in enumerate((_Q0, _Q0 + _Q1, _Q0 + _Q1 + _Q0)):
    cnt = jnp.sum((d < thr).astype(jnp.int32))
    cnts = jnp.where(lane == k, cnt, cnts)
  split_ref[...] = cnts
  dstp_ref[0:_NEB, :] = d
  dstp_ref[_NEB:, :] = jnp.zeros((_NEBP - _NEB, _EB), jnp.int32)


def _tc_edge_body(ed0_ref, ed1_ref, ed2_ref, w_ref, a2w_ref, mp_ref, a2_ref):
  mp = jnp.dot(ed0_ref[0], w_ref[0], preferred_element_type=jnp.float32)
  mp += jnp.dot(ed1_ref[0], w_ref[1], preferred_element_type=jnp.float32)
  mp += jnp.dot(ed2_ref[0], w_ref[2], preferred_element_type=jnp.float32)
  mp_ref[...] = mp
  a2_ref[...] = jax.lax.dot_general(
      a2w_ref[...], mp, (((1,), (1,)), ((), ())),
      preferred_element_type=jnp.float32)


def _sc_body(mp_hbm, a2_hbm, dst_hbm, a1_hbm, split_hbm, out_hbm,
             a1_v, split_v, dst_v, a2h0_v, a2h1_v, mp_v, w0_v, w1_v, idx_v,
             vals0_v, vals1_v, den_v, sem_in, sem_sc, s0_sh, s1_sh, d_sh):
  c = lax.axis_index("c")
  s = lax.axis_index("s")

  pltpu.sync_copy(split_hbm, split_v)
  pltpu.sync_copy(a1_hbm, a1_v)

  sp = split_v[pl.ds(0, 16)]
  sp0 = sp[0]
  sp1 = sp[1]
  sp2 = sp[2]

  iota = lax.iota(jnp.int32, 16)
  oh0 = (iota == 0).astype(jnp.float32)
  oh1 = (iota == 1).astype(jnp.float32)

  def zden(r, carry):
    for k in range(_D // 16):
      den_v[r, pl.ds(16 * k, 16)] = jnp.zeros((16,), jnp.float32)
    return carry
  lax.fori_loop(0, _C, zden, 0)

  for p in range(2):
    if p == 0:
      estart = jnp.where(c == 0, 0, sp1)
      eend = jnp.where(c == 0, sp0, sp2)
      node_base = c * (_Q0 + _Q1)
      qsize = _Q0
    else:
      estart = jnp.where(c == 0, sp0, sp2)
      eend = jnp.where(c == 0, sp1, _E)
      node_base = _Q0 + c * (_Q0 + _Q1)
      qsize = _Q1

    def zrow(r, carry):
      for k in range(_D // 16):
        vals0_v[r, pl.ds(16 * k, 16)] = jnp.zeros((16,), jnp.float32)
      return carry
    lax.fori_loop(0, 16, zrow, 0)
    row0 = pl.multiple_of(s * _RPT, 8)
    for j in range(_RPT // 16):
      pltpu.sync_copy(vals0_v.at[pl.ds(0, 16), :],
                      s0_sh.at[pl.ds(row0 + j * 16, 16), :])
      pltpu.sync_copy(vals0_v.at[pl.ds(0, 16), :],
                      s1_sh.at[pl.ds(row0 + j * 16, 16), :])
      pltpu.sync_copy(vals0_v.at[pl.ds(0, 16), :],
                      d_sh.at[pl.ds(row0 + j * 16, 16), :])
    plsc.subcore_barrier()

    astart = (estart // 128) * 128
    cnt = eend - astart
    per = ((cnt + 2047) // 2048) * 128
    tstart = astart + s * per
    tend = jnp.minimum(tstart + per, eend)
    nch = (jnp.maximum(tend - tstart, 0) + (_C - 1)) // _C

    def chunk_body(j, carry):
      base = pl.multiple_of(tstart + j * _C, 64)
      d_dst = pltpu.async_copy(dst_hbm.at[pl.ds(base, _C)], dst_v, sem_in)
      d_mp = pltpu.async_copy(mp_hbm.at[pl.ds(base, _C), :], mp_v, sem_in)

      @pl.when(j % 2 == 0)
      def _fetch_a2():
        abase = pl.multiple_of(base, 128)
        pltpu.sync_copy(a2_hbm.at[0, pl.ds(abase, 2 * _C)], a2h0_v)
        pltpu.sync_copy(a2_hbm.at[1, pl.ds(abase, 2 * _C)], a2h1_v)

      @pl.when(j > 0)
      def _drain():
        pltpu.make_async_copy(vals0_v, s0_sh.at[idx_v], sem_sc).wait()
        pltpu.make_async_copy(vals1_v, s1_sh.at[idx_v], sem_sc).wait()
        pltpu.make_async_copy(den_v, d_sh.at[idx_v], sem_sc).wait()

      d_dst.wait()
      d_mp.wait()
      for g in range(_G):
        dstv = dst_v[pl.ds(g * 16, 16)]
        eidx = base + g * 16 + iota
        mask = (eidx >= estart) & (eidx < tend)
        a1h0 = plsc.load_gather(a1_v, [2 * dstv])
        a1h1 = plsc.load_gather(a1_v, [2 * dstv + 1])
        apos = (j % 2) * _C + g * 16 + iota
        a2h0 = plsc.load_gather(a2h0_v, [apos])
        a2h1 = plsc.load_gather(a2h1_v, [apos])
        x0 = a1h0 + a2h0
        x0 = jnp.where(x0 > 0, x0, 0.01 * x0)
        x1 = a1h1 + a2h1
        x1 = jnp.where(x1 > 0, x1, 0.01 * x1)
        w0 = jnp.where(mask, jnp.exp(x0), 0.0)
        w1 = jnp.where(mask, jnp.exp(x1), 0.0)
        w0_v[pl.ds(g * 16, 16)] = w0
        w1_v[pl.ds(g * 16, 16)] = w1
        idx_v[pl.ds(g * 16, 16)] = jnp.where(mask, dstv - node_base, _DUMP)

      def edge_group(g2, carry2):
        wv0 = w0_v[pl.ds(g2 * 16, 16)]
        wv1 = w1_v[pl.ds(g2 * 16, 16)]
        for l in range(16):
          e = g2 * 16 + l
          w0 = wv0[l]
          w1 = wv1[l]
          for k in range(_D // 16):
            m = mp_v[e, pl.ds(16 * k, 16)]
            vals0_v[e, pl.ds(16 * k, 16)] = m * w0
            vals1_v[e, pl.ds(16 * k, 16)] = m * w1
          den_v[e, pl.ds(0, 16)] = oh0 * w0 + oh1 * w1
        return carry2
      lax.fori_loop(0, _G, edge_group, 0)

      pltpu.async_copy(vals0_v, s0_sh.at[idx_v], sem_sc, add=True)
      pltpu.async_copy(vals1_v, s1_sh.at[idx_v], sem_sc, add=True)
      pltpu.async_copy(den_v, d_sh.at[idx_v], sem_sc, add=True)
      return carry
    lax.fori_loop(0, nch, chunk_body, 0)

    @pl.when(nch > 0)
    def _final_drain():
      pltpu.make_async_copy(vals0_v, s0_sh.at[idx_v], sem_sc).wait()
      pltpu.make_async_copy(vals1_v, s1_sh.at[idx_v], sem_sc).wait()
      pltpu.make_async_copy(den_v, d_sh.at[idx_v], sem_sc).wait()

    plsc.subcore_barrier()

    rstart = s * _RPT
    rend = jnp.minimum(rstart + _RPT, qsize)
    rcnt = jnp.maximum(rend - rstart, 0)
    nch2 = (rcnt + _OB - 1) // _OB

    def out_body(ch, carry):
      r0 = pl.multiple_of(
          jnp.minimum(rstart + ch * _OB, rend - _OB), 8)
      pltpu.sync_copy(s0_sh.at[pl.ds(r0, _OB), :], vals0_v.at[pl.ds(0, _OB), :])
      pltpu.sync_copy(s1_sh.at[pl.ds(r0, _OB), :], vals1_v.at[pl.ds(0, _OB), :])
      pltpu.sync_copy(d_sh.at[pl.ds(r0, _OB), :], den_v.at[pl.ds(0, _OB), :])

      def row_body(r, carry2):
        den = den_v[r, pl.ds(0, 16)]
        invv = 1.0 / jnp.maximum(den, 1e-20)
        inv0 = invv[0]
        inv1 = invv[1]
        for k in range(_D // 16):
          x = vals0_v[r, pl.ds(16 * k, 16)] * inv0
          vals0_v[r, pl.ds(16 * k, 16)] = jnp.where(x > 0, x, jnp.exp(x) - 1.0)
        for k in range(_D // 16):
          x = vals1_v[r, pl.ds(16 * k, 16)] * inv1
          vals1_v[r, pl.ds(16 * k, 16)] = jnp.where(x > 0, x, jnp.exp(x) - 1.0)
        return carry2
      lax.fori_loop(0, _OB, row_body, 0)

      obase = pl.multiple_of(node_base + r0, 8)
      pltpu.sync_copy(vals0_v.at[pl.ds(0, _OB), :],
                      out_hbm.at[pl.ds(obase, _OB), pl.ds(0, _D)])
      pltpu.sync_copy(vals1_v.at[pl.ds(0, _OB), :],
                      out_hbm.at[pl.ds(obase, _OB), pl.ds(_D, _D)])
      return carry
    lax.fori_loop(0, nch2, out_body, 0)

    if p == 0:
      lax.fori_loop(0, _OB, zden, 0)
      plsc.subcore_barrier()


_sc_kernel = functools.partial(
    pl.kernel,
    out_type=jax.ShapeDtypeStruct((_N, 2 * _D), jnp.float32),
    mesh=plsc.VectorSubcoreMesh(core_axis_name="c", subcore_axis_name="s"),
    compiler_params=pltpu.CompilerParams(needs_layout_passes=False),
    scratch_types=[
        pltpu.VMEM((2 * _N,), jnp.float32),
        pltpu.VMEM((16,), jnp.int32),
        pltpu.VMEM((_C,), jnp.int32),
        pltpu.VMEM((2 * _C,), jnp.float32),
        pltpu.VMEM((2 * _C,), jnp.float32),
        pltpu.VMEM((_C, _D), jnp.float32),
        pltpu.VMEM((_C,), jnp.float32),
        pltpu.VMEM((_C,), jnp.float32),
        pltpu.VMEM((_C,), jnp.int32),
        pltpu.VMEM((_C, _D), jnp.float32),
        pltpu.VMEM((_C, _D), jnp.float32),
        pltpu.VMEM((_C, _D), jnp.float32),
        pltpu.SemaphoreType.DMA,
        pltpu.SemaphoreType.DMA,
        pltpu.VMEM_SHARED((_NROWS, _D), jnp.float32),
        pltpu.VMEM_SHARED((_NROWS, _D), jnp.float32),
        pltpu.VMEM_SHARED((_NROWS, _D), jnp.float32),
    ],
)(_sc_body)


def _build_rot_weight(r_vec):
  rv = r_vec / jnp.maximum(jnp.linalg.norm(r_vec, axis=2, keepdims=True), 1e-12)
  rv2 = jnp.stack([rv, rv], axis=1)
  rv2 = rv2.at[:, 1, :, 1].set(-rv2[:, 1, :, 1])
  rv2 = rv2.reshape(r_vec.shape[0] * 2, _D // 2, 2)
  final = jnp.zeros((_L, _D // 2, 2), jnp.float32)
  final = final.at[-1, :, 0].set(1.0)
  for i in range(_L - 2, -1, -1):
    re = final[i + 1, :, 0] * rv2[_ETYPES[i], :, 0] - final[i + 1, :, 1] * rv2[_ETYPES[i], :, 1]
    im = final[i + 1, :, 0] * rv2[_ETYPES[i], :, 1] + final[i + 1, :, 1] * rv2[_ETYPES[i], :, 0]
    final = final.at[i, :, 0].set(re)
    final = final.at[i, :, 1].set(im)
  cc = final[:, :, 0]
  ss = final[:, :, 1]
  m = jnp.stack([jnp.stack([cc, ss], -1), jnp.stack([-ss, cc], -1)], -2) / 3.0
  return jnp.einsum("pq,ipab->ipaqb", jnp.eye(_D // 2, dtype=jnp.float32),
                    m).reshape(_L * _D, _D)


def kernel(features, edata, dst_index, attn1_w, attn2, r_vec,
           fusion_w1, fusion_b1, fusion_w2):
  del fusion_w1, fusion_b1, fusion_w2
  w_rot = _build_rot_weight(r_vec).reshape(_L, _D, _D)
  edata_t = jnp.transpose(edata, (1, 0, 2))
  dst2d = dst_index.astype(jnp.int32).reshape(_NEB, _EB)

  a1n, splitv, dstp2d = pl.pallas_call(
      _tc_node_body,
      out_shape=[
          jax.ShapeDtypeStruct((_N, 2), jnp.float32),
          jax.ShapeDtypeStruct((1, 16), jnp.int32),
          jax.ShapeDtypeStruct((_EPAD // _EB, _EB), jnp.int32),
      ],
  )(features, attn1_w, dst2d)

  mp, a2 = pl.pallas_call(
      _tc_edge_body,
      grid=(_NEBP,),
      in_specs=[
          pl.BlockSpec((1, _EB, _D),
                       lambda i: (0, jnp.minimum(i, _NEB - 1), 0)),
          pl.BlockSpec((1, _EB, _D),
                       lambda i: (1, jnp.minimum(i, _NEB - 1), 0)),
          pl.BlockSpec((1, _EB, _D),
                       lambda i: (2, jnp.minimum(i, _NEB - 1), 0)),
          pl.BlockSpec((_L, _D, _D), lambda i: (0, 0, 0)),
          pl.BlockSpec((_H, _D), lambda i: (0, 0)),
      ],
      out_specs=[
          pl.BlockSpec((_EB, _D), lambda i: (i, 0)),
          pl.BlockSpec((2, _EB), lambda i: (0, i)),
      ],
      out_shape=[
          jax.ShapeDtypeStruct((_EPAD, _D), jnp.float32),
          jax.ShapeDtypeStruct((2, _EPAD), jnp.float32),
      ],
  )(edata_t, edata_t, edata_t, w_rot, attn2[0])

  return _sc_kernel(mp, a2, dstp2d.reshape(-1), a1n.reshape(-1),
                    splitv.reshape(-1))


# speedup vs baseline: 1.9114x; 1.4541x over previous
import functools

import jax
import jax.numpy as jnp
from jax import lax
from jax.experimental import pallas as pl
from jax.experimental.pallas import tpu as pltpu
from jax.experimental.pallas import tpu_sc as plsc

_N = 10000
_E = 160000
_L = 3
_D = 128
_H = 2
_ETYPES = (1, 3)

_EB = 640
_NEB = _E // _EB
_NEBP = 254
_EPAD = _NEBP * _EB

_C = 64
_G = _C // 16
_Q0 = 2504
_Q1 = 2496
_DUMP = 2520
_RPT = 160
_NROWS = 16 * _RPT
_OB = 40


def _tc_node_body(feat_ref, w1_ref, dst_ref, a1n_ref, split_ref, dstp_ref):
  f = feat_ref[...]
  s0 = jnp.sum(f * w1_ref[0:1, :], axis=1, keepdims=True)
  s1 = jnp.sum(f * w1_ref[1:2, :], axis=1, keepdims=True)
  a1n_ref[...] = jnp.concatenate([s0, s1], axis=1)
  d = dst_ref[...]
  lane = lax.broadcasted_iota(jnp.int32, (1, 16), 1)
  cnts = jnp.zeros((1, 16), jnp.int32)
  for k, thr in enumerate((_Q0, _Q0 + _Q1, _Q0 + _Q1 + _Q0)):
    cnt = jnp.sum((d < thr).astype(jnp.int32))
    cnts = jnp.where(lane == k, cnt, cnts)
  split_ref[...] = cnts
  dstp_ref[0:_NEB, :] = d
  dstp_ref[_NEB:, :] = jnp.zeros((_NEBP - _NEB, _EB), jnp.int32)


def _tc_edge_body(ed0_ref, ed1_ref, ed2_ref, w_ref, a2w_ref, mp_ref, a2_ref):
  mp = jnp.dot(ed0_ref[0], w_ref[0], preferred_element_type=jnp.float32)
  mp += jnp.dot(ed1_ref[0], w_ref[1], preferred_element_type=jnp.float32)
  mp += jnp.dot(ed2_ref[0], w_ref[2], preferred_element_type=jnp.float32)
  mp_ref[...] = mp
  a2_ref[...] = jax.lax.dot_general(
      a2w_ref[...], mp, (((1,), (1,)), ((), ())),
      preferred_element_type=jnp.float32)


def _sc_body(mp_hbm, a2_hbm, dst_hbm, a1_hbm, split_hbm, out_hbm,
             a1_v, split_v, dst_v, a2h0_v, a2h1_v, mp_v, w0_v, w1_v, idx_v,
             vals0_v, vals1_v, den_v, sem_in, sem_sc, s0_sh, s1_sh, d_sh):
  c = lax.axis_index("c")
  s = lax.axis_index("s")

  pltpu.sync_copy(split_hbm, split_v)
  pltpu.sync_copy(a1_hbm, a1_v)

  sp = split_v[pl.ds(0, 16)]
  sp0 = sp[0]
  sp1 = sp[1]
  sp2 = sp[2]

  iota = lax.iota(jnp.int32, 16)
  oh0 = (iota == 0).astype(jnp.float32)
  oh1 = (iota == 1).astype(jnp.float32)

  def zden(r, carry):
    for k in range(_D // 16):
      den_v[r, pl.ds(16 * k, 16)] = jnp.zeros((16,), jnp.float32)
    return carry
  lax.fori_loop(0, _C, zden, 0)

  for p in range(2):
    if p == 0:
      estart = jnp.where(c == 0, 0, sp1)
      eend = jnp.where(c == 0, sp0, sp2)
      node_base = c * (_Q0 + _Q1)
      qsize = _Q0
    else:
      estart = jnp.where(c == 0, sp0, sp2)
      eend = jnp.where(c == 0, sp1, _E)
      node_base = _Q0 + c * (_Q0 + _Q1)
      qsize = _Q1

    def zrow(r, carry):
      for k in range(_D // 16):
        vals0_v[r, pl.ds(16 * k, 16)] = jnp.zeros((16,), jnp.float32)
      return carry
    lax.fori_loop(0, 16, zrow, 0)
    row0 = pl.multiple_of(s * _RPT, 8)
    for j in range(_RPT // 16):
      pltpu.sync_copy(vals0_v.at[pl.ds(0, 16), :],
                      s0_sh.at[pl.ds(row0 + j * 16, 16), :])
      pltpu.sync_copy(vals0_v.at[pl.ds(0, 16), :],
                      s1_sh.at[pl.ds(row0 + j * 16, 16), :])
      pltpu.sync_copy(vals0_v.at[pl.ds(0, 16), :],
                      d_sh.at[pl.ds(row0 + j * 16, 16), :])
    plsc.subcore_barrier()

    astart = (estart // 128) * 128
    cnt = eend - astart
    per = ((cnt + 2047) // 2048) * 128
    tstart = astart + s * per
    tend = jnp.minimum(tstart + per, eend)
    nch = (jnp.maximum(tend - tstart, 0) + (_C - 1)) // _C

    def chunk_body(j, carry):
      base = pl.multiple_of(tstart + j * _C, 64)
      d_dst = pltpu.async_copy(dst_hbm.at[pl.ds(base, _C)], dst_v, sem_in)
      d_mp = pltpu.async_copy(mp_hbm.at[pl.ds(base, _C), :], mp_v, sem_in)

      @pl.when(j % 2 == 0)
      def _fetch_a2():
        abase = pl.multiple_of(base, 128)
        pltpu.sync_copy(a2_hbm.at[0, pl.ds(abase, 2 * _C)], a2h0_v)
        pltpu.sync_copy(a2_hbm.at[1, pl.ds(abase, 2 * _C)], a2h1_v)

      @pl.when(j > 0)
      def _drain():
        pltpu.make_async_copy(vals0_v, s0_sh.at[idx_v], sem_sc).wait()
        pltpu.make_async_copy(vals1_v, s1_sh.at[idx_v], sem_sc).wait()
        pltpu.make_async_copy(den_v, d_sh.at[idx_v], sem_sc).wait()

      d_dst.wait()
      d_mp.wait()
      for g in range(_G):
        dstv = dst_v[pl.ds(g * 16, 16)]
        eidx = base + g * 16 + iota
        mask = (eidx >= estart) & (eidx < tend)
        a1h0 = plsc.load_gather(a1_v, [2 * dstv])
        a1h1 = plsc.load_gather(a1_v, [2 * dstv + 1])
        apos = (j % 2) * _C + g * 16 + iota
        a2h0 = plsc.load_gather(a2h0_v, [apos])
        a2h1 = plsc.load_gather(a2h1_v, [apos])
        x0 = a1h0 + a2h0
        x0 = jnp.where(x0 > 0, x0, 0.01 * x0)
        x1 = a1h1 + a2h1
        x1 = jnp.where(x1 > 0, x1, 0.01 * x1)
        w0 = jnp.where(mask, jnp.exp(x0), 0.0)
        w1 = jnp.where(mask, jnp.exp(x1), 0.0)
        w0_v[pl.ds(g * 16, 16)] = w0
        w1_v[pl.ds(g * 16, 16)] = w1
        idx_v[pl.ds(g * 16, 16)] = jnp.where(mask, dstv - node_base, _DUMP)

      def edge_group(g2, carry2):
        wv0 = w0_v[pl.ds(g2 * 16, 16)]
        wv1 = w1_v[pl.ds(g2 * 16, 16)]
        for l in range(16):
          e = g2 * 16 + l
          w0 = wv0[l]
          w1 = wv1[l]
          for k in range(_D // 16):
            m = mp_v[e, pl.ds(16 * k, 16)]
            vals0_v[e, pl.ds(16 * k, 16)] = m * w0
            vals1_v[e, pl.ds(16 * k, 16)] = m * w1
          den_v[e, pl.ds(0, 16)] = oh0 * w0 + oh1 * w1
        return carry2
      lax.fori_loop(0, _G, edge_group, 0)

      pltpu.async_copy(vals0_v, s0_sh.at[idx_v], sem_sc, add=True)
      pltpu.async_copy(vals1_v, s1_sh.at[idx_v], sem_sc, add=True)
      pltpu.async_copy(den_v, d_sh.at[idx_v], sem_sc, add=True)
      return carry
    lax.fori_loop(0, nch, chunk_body, 0)

    @pl.when(nch > 0)
    def _final_drain():
      pltpu.make_async_copy(vals0_v, s0_sh.at[idx_v], sem_sc).wait()
      pltpu.make_async_copy(vals1_v, s1_sh.at[idx_v], sem_sc).wait()
      pltpu.make_async_copy(den_v, d_sh.at[idx_v], sem_sc).wait()

    plsc.subcore_barrier()

    rstart = s * _RPT
    rend = jnp.minimum(rstart + _RPT, qsize)
    rcnt = jnp.maximum(rend - rstart, 0)
    nch2 = (rcnt + _OB - 1) // _OB

    def out_body(ch, carry):
      r0 = pl.multiple_of(
          jnp.minimum(rstart + ch * _OB, rend - _OB), 8)
      pltpu.sync_copy(s0_sh.at[pl.ds(r0, _OB), :], vals0_v.at[pl.ds(0, _OB), :])
      pltpu.sync_copy(s1_sh.at[pl.ds(r0, _OB), :], vals1_v.at[pl.ds(0, _OB), :])
      pltpu.sync_copy(d_sh.at[pl.ds(r0, _OB), :], den_v.at[pl.ds(0, _OB), :])

      def row_body(r, carry2):
        den = den_v[r, pl.ds(0, 16)]
        invv = 1.0 / jnp.maximum(den, 1e-20)
        inv0 = invv[0]
        inv1 = invv[1]
        for k in range(_D // 16):
          x = vals0_v[r, pl.ds(16 * k, 16)] * inv0
          vals0_v[r, pl.ds(16 * k, 16)] = jnp.where(x > 0, x, jnp.exp(x) - 1.0)
        for k in range(_D // 16):
          x = vals1_v[r, pl.ds(16 * k, 16)] * inv1
          vals1_v[r, pl.ds(16 * k, 16)] = jnp.where(x > 0, x, jnp.exp(x) - 1.0)
        return carry2
      lax.fori_loop(0, _OB, row_body, 0)

      obase = pl.multiple_of(node_base + r0, 8)
      pltpu.sync_copy(vals0_v.at[pl.ds(0, _OB), :],
                      out_hbm.at[pl.ds(obase, _OB), pl.ds(0, _D)])
      pltpu.sync_copy(vals1_v.at[pl.ds(0, _OB), :],
                      out_hbm.at[pl.ds(obase, _OB), pl.ds(_D, _D)])
      return carry
    lax.fori_loop(0, nch2, out_body, 0)

    if p == 0:
      lax.fori_loop(0, _OB, zden, 0)
      plsc.subcore_barrier()


_sc_kernel = functools.partial(
    pl.kernel,
    out_type=jax.ShapeDtypeStruct((_N, 2 * _D), jnp.float32),
    mesh=plsc.VectorSubcoreMesh(core_axis_name="c", subcore_axis_name="s"),
    compiler_params=pltpu.CompilerParams(needs_layout_passes=False),
    scratch_types=[
        pltpu.VMEM((2 * _N,), jnp.float32),
        pltpu.VMEM((16,), jnp.int32),
        pltpu.VMEM((_C,), jnp.int32),
        pltpu.VMEM((2 * _C,), jnp.float32),
        pltpu.VMEM((2 * _C,), jnp.float32),
        pltpu.VMEM((_C, _D), jnp.float32),
        pltpu.VMEM((_C,), jnp.float32),
        pltpu.VMEM((_C,), jnp.float32),
        pltpu.VMEM((_C,), jnp.int32),
        pltpu.VMEM((_C, _D), jnp.float32),
        pltpu.VMEM((_C, _D), jnp.float32),
        pltpu.VMEM((_C, _D), jnp.float32),
        pltpu.SemaphoreType.DMA,
        pltpu.SemaphoreType.DMA,
        pltpu.VMEM_SHARED((_NROWS, _D), jnp.float32),
        pltpu.VMEM_SHARED((_NROWS, _D), jnp.float32),
        pltpu.VMEM_SHARED((_NROWS, _D), jnp.float32),
    ],
)(_sc_body)


def _build_rot_weight(r_vec):
  rv = r_vec / jnp.maximum(jnp.linalg.norm(r_vec, axis=2, keepdims=True), 1e-12)
  rv2 = jnp.stack([rv, rv], axis=1)
  rv2 = rv2.at[:, 1, :, 1].set(-rv2[:, 1, :, 1])
  rv2 = rv2.reshape(r_vec.shape[0] * 2, _D // 2, 2)
  final = jnp.zeros((_L, _D // 2, 2), jnp.float32)
  final = final.at[-1, :, 0].set(1.0)
  for i in range(_L - 2, -1, -1):
    re = final[i + 1, :, 0] * rv2[_ETYPES[i], :, 0] - final[i + 1, :, 1] * rv2[_ETYPES[i], :, 1]
    im = final[i + 1, :, 0] * rv2[_ETYPES[i], :, 1] + final[i + 1, :, 1] * rv2[_ETYPES[i], :, 0]
    final = final.at[i, :, 0].set(re)
    final = final.at[i, :, 1].set(im)
  cc = final[:, :, 0]
  ss = final[:, :, 1]
  m = jnp.stack([jnp.stack([cc, ss], -1), jnp.stack([-ss, cc], -1)], -2) / 3.0
  return jnp.einsum("pq,ipab->ipaqb", jnp.eye(_D // 2, dtype=jnp.float32),
                    m).reshape(_L * _D, _D)


def kernel(features, edata, dst_index, attn1_w, attn2, r_vec,
           fusion_w1, fusion_b1, fusion_w2):
  del fusion_w1, fusion_b1, fusion_w2
  w_rot = _build_rot_weight(r_vec).reshape(_L, _D, _D)
  edata_t = jnp.transpose(edata, (1, 0, 2))
  dst2d = dst_index.astype(jnp.int32).reshape(_NEB, _EB)

  a1n, splitv, dstp2d = pl.pallas_call(
      _tc_node_body,
      out_shape=[
          jax.ShapeDtypeStruct((_N, 2), jnp.float32),
          jax.ShapeDtypeStruct((1, 16), jnp.int32),
          jax.ShapeDtypeStruct((_EPAD // _EB, _EB), jnp.int32),
      ],
  )(features, attn1_w, dst2d)

  mp, a2 = pl.pallas_call(
      _tc_edge_body,
      grid=(_NEBP,),
      in_specs=[
          pl.BlockSpec((1, _EB, _D),
                       lambda i: (0, jnp.minimum(i, _NEB - 1), 0)),
          pl.BlockSpec((1, _EB, _D),
                       lambda i: (1, jnp.minimum(i, _NEB - 1), 0)),
          pl.BlockSpec((1, _EB, _D),
                       lambda i: (2, jnp.minimum(i, _NEB - 1), 0)),
          pl.BlockSpec((_L, _D, _D), lambda i: (0, 0, 0)),
          pl.BlockSpec((_H, _D), lambda i: (0, 0)),
      ],
      out_specs=[
          pl.BlockSpec((_EB, _D), lambda i: (i, 0)),
          pl.BlockSpec((2, _EB), lambda i: (0, i)),
      ],
      out_shape=[
          jax.ShapeDtypeStruct((_EPAD, _D), jnp.float32),
          jax.ShapeDtypeStruct((2, _EPAD), jnp.float32),
      ],
  )(edata_t, edata_t, edata_t, w_rot, attn2[0])

  return _sc_kernel(mp, a2, dstp2d.reshape(-1), a1n.reshape(-1),
                    splitv.reshape(-1))


# speedup vs baseline: 2.3274x; 1.2176x over previous
import functools

import jax
import jax.numpy as jnp
from jax import lax
from jax.experimental import pallas as pl
from jax.experimental.pallas import tpu as pltpu
from jax.experimental.pallas import tpu_sc as plsc

_N = 10000
_E = 160000
_L = 3
_D = 128
_H = 2
_ETYPES = (1, 3)

_EB = 1280
_NEB = _E // _EB
_NEBP = 128
_EPAD = _NEBP * _EB

_C = 64
_G = _C // 16
_Q0 = 2504
_Q1 = 2496
_DUMP = 2520
_RPT = 160
_NROWS = 16 * _RPT
_OB = 40


def _tc_node_body(feat_ref, w1_ref, dst_ref, a1n_ref, split_ref, dstp_ref):
  f = feat_ref[...]
  s0 = jnp.sum(f * w1_ref[0:1, :], axis=1, keepdims=True)
  s1 = jnp.sum(f * w1_ref[1:2, :], axis=1, keepdims=True)
  a1n_ref[...] = jnp.concatenate([s0, s1], axis=1)
  d = dst_ref[...]
  lane = lax.broadcasted_iota(jnp.int32, (1, 16), 1)
  cnts = jnp.zeros((1, 16), jnp.int32)
  for k, thr in enumerate((_Q0, _Q0 + _Q1, _Q0 + _Q1 + _Q0)):
    cnt = jnp.sum((d < thr).astype(jnp.int32))
    cnts = jnp.where(lane == k, cnt, cnts)
  split_ref[...] = cnts
  dstp_ref[0:_NEB, :] = d
  dstp_ref[_NEB:, :] = jnp.zeros((_NEBP - _NEB, _EB), jnp.int32)


def _tc_edge_body(ed0_ref, ed1_ref, ed2_ref, w_ref, a2w_ref, mp_ref, a2_ref):
  mp = jnp.dot(ed0_ref[0], w_ref[0], preferred_element_type=jnp.float32)
  mp += jnp.dot(ed1_ref[0], w_ref[1], preferred_element_type=jnp.float32)
  mp += jnp.dot(ed2_ref[0], w_ref[2], preferred_element_type=jnp.float32)
  mp_ref[...] = mp
  a2_ref[...] = jax.lax.dot_general(
      a2w_ref[...], mp, (((1,), (1,)), ((), ())),
      preferred_element_type=jnp.float32)


def _sc_body(mp_hbm, a2_hbm, dst_hbm, a1_hbm, split_hbm, out_hbm,
             a1_v, split_v, dst_v, a2h0_v, a2h1_v, mp_v, w0_v, w1_v, idx_v,
             vals0_v, vals1_v, den_v, sem_in, sem_sc, s0_sh, s1_sh, d_sh):
  c = lax.axis_index("c")
  s = lax.axis_index("s")

  pltpu.sync_copy(split_hbm, split_v)
  pltpu.sync_copy(a1_hbm, a1_v)

  sp = split_v[pl.ds(0, 16)]
  sp0 = sp[0]
  sp1 = sp[1]
  sp2 = sp[2]

  iota = lax.iota(jnp.int32, 16)
  oh0 = (iota == 0).astype(jnp.float32)
  oh1 = (iota == 1).astype(jnp.float32)

  def zden(r, carry):
    for k in range(_D // 16):
      den_v[r, pl.ds(16 * k, 16)] = jnp.zeros((16,), jnp.float32)
    return carry
  lax.fori_loop(0, _C, zden, 0)

  for p in range(2):
    if p == 0:
      estart = jnp.where(c == 0, 0, sp1)
      eend = jnp.where(c == 0, sp0, sp2)
      node_base = c * (_Q0 + _Q1)
      qsize = _Q0
    else:
      estart = jnp.where(c == 0, sp0, sp2)
      eend = jnp.where(c == 0, sp1, _E)
      node_base = _Q0 + c * (_Q0 + _Q1)
      qsize = _Q1

    def zrow(r, carry):
      for k in range(_D // 16):
        vals0_v[r, pl.ds(16 * k, 16)] = jnp.zeros((16,), jnp.float32)
      return carry
    lax.fori_loop(0, 16, zrow, 0)
    row0 = pl.multiple_of(s * _RPT, 8)
    for j in range(_RPT // 16):
      pltpu.sync_copy(vals0_v.at[pl.ds(0, 16), :],
                      s0_sh.at[pl.ds(row0 + j * 16, 16), :])
      pltpu.sync_copy(vals0_v.at[pl.ds(0, 16), :],
                      s1_sh.at[pl.ds(row0 + j * 16, 16), :])
      pltpu.sync_copy(vals0_v.at[pl.ds(0, 16), :],
                      d_sh.at[pl.ds(row0 + j * 16, 16), :])
    plsc.subcore_barrier()

    astart = (estart // 128) * 128
    cnt = eend - astart
    per = ((cnt + 2047) // 2048) * 128
    tstart = astart + s * per
    tend = jnp.minimum(tstart + per, eend)
    nch = (jnp.maximum(tend - tstart, 0) + (_C - 1)) // _C

    def chunk_body(j, carry):
      base = pl.multiple_of(tstart + j * _C, 64)
      d_dst = pltpu.async_copy(dst_hbm.at[pl.ds(base, _C)], dst_v, sem_in)
      d_mp = pltpu.async_copy(mp_hbm.at[pl.ds(base, _C), :], mp_v, sem_in)

      @pl.when(j % 2 == 0)
      def _fetch_a2():
        abase = pl.multiple_of(base, 128)
        pltpu.sync_copy(a2_hbm.at[0, pl.ds(abase, 2 * _C)], a2h0_v)
        pltpu.sync_copy(a2_hbm.at[1, pl.ds(abase, 2 * _C)], a2h1_v)

      @pl.when(j > 0)
      def _drain():
        pltpu.make_async_copy(vals0_v, s0_sh.at[idx_v], sem_sc).wait()
        pltpu.make_async_copy(vals1_v, s1_sh.at[idx_v], sem_sc).wait()
        pltpu.make_async_copy(den_v, d_sh.at[idx_v], sem_sc).wait()

      d_dst.wait()
      d_mp.wait()
      for g in range(_G):
        dstv = dst_v[pl.ds(g * 16, 16)]
        eidx = base + g * 16 + iota
        mask = (eidx >= estart) & (eidx < tend)
        a1h0 = plsc.load_gather(a1_v, [2 * dstv])
        a1h1 = plsc.load_gather(a1_v, [2 * dstv + 1])
        apos = (j % 2) * _C + g * 16 + iota
        a2h0 = plsc.load_gather(a2h0_v, [apos])
        a2h1 = plsc.load_gather(a2h1_v, [apos])
        x0 = a1h0 + a2h0
        x0 = jnp.where(x0 > 0, x0, 0.01 * x0)
        x1 = a1h1 + a2h1
        x1 = jnp.where(x1 > 0, x1, 0.01 * x1)
        w0 = jnp.where(mask, jnp.exp(x0), 0.0)
        w1 = jnp.where(mask, jnp.exp(x1), 0.0)
        w0_v[pl.ds(g * 16, 16)] = w0
        w1_v[pl.ds(g * 16, 16)] = w1
        idx_v[pl.ds(g * 16, 16)] = jnp.where(mask, dstv - node_base, _DUMP)

      def edge_group(g2, carry2):
        wv0 = w0_v[pl.ds(g2 * 16, 16)]
        wv1 = w1_v[pl.ds(g2 * 16, 16)]
        for l in range(16):
          e = g2 * 16 + l
          w0 = wv0[l]
          w1 = wv1[l]
          for k in range(_D // 16):
            m = mp_v[e, pl.ds(16 * k, 16)]
            vals0_v[e, pl.ds(16 * k, 16)] = m * w0
            vals1_v[e, pl.ds(16 * k, 16)] = m * w1
          den_v[e, pl.ds(0, 16)] = oh0 * w0 + oh1 * w1
        return carry2
      lax.fori_loop(0, _G, edge_group, 0)

      pltpu.async_copy(vals0_v, s0_sh.at[idx_v], sem_sc, add=True)
      pltpu.async_copy(vals1_v, s1_sh.at[idx_v], sem_sc, add=True)
      pltpu.async_copy(den_v, d_sh.at[idx_v], sem_sc, add=True)
      return carry
    lax.fori_loop(0, nch, chunk_body, 0)

    @pl.when(nch > 0)
    def _final_drain():
      pltpu.make_async_copy(vals0_v, s0_sh.at[idx_v], sem_sc).wait()
      pltpu.make_async_copy(vals1_v, s1_sh.at[idx_v], sem_sc).wait()
      pltpu.make_async_copy(den_v, d_sh.at[idx_v], sem_sc).wait()

    plsc.subcore_barrier()

    rstart = s * _RPT
    rend = jnp.minimum(rstart + _RPT, qsize)
    rcnt = jnp.maximum(rend - rstart, 0)
    nch2 = (rcnt + _OB - 1) // _OB

    def out_body(ch, carry):
      r0 = pl.multiple_of(
          jnp.minimum(rstart + ch * _OB, rend - _OB), 8)
      pltpu.sync_copy(s0_sh.at[pl.ds(r0, _OB), :], vals0_v.at[pl.ds(0, _OB), :])
      pltpu.sync_copy(s1_sh.at[pl.ds(r0, _OB), :], vals1_v.at[pl.ds(0, _OB), :])
      pltpu.sync_copy(d_sh.at[pl.ds(r0, _OB), :], den_v.at[pl.ds(0, _OB), :])

      def row_body(r, carry2):
        den = den_v[r, pl.ds(0, 16)]
        invv = 1.0 / jnp.maximum(den, 1e-20)
        inv0 = invv[0]
        inv1 = invv[1]
        for k in range(_D // 16):
          x = vals0_v[r, pl.ds(16 * k, 16)] * inv0
          vals0_v[r, pl.ds(16 * k, 16)] = jnp.where(x > 0, x, jnp.exp(x) - 1.0)
        for k in range(_D // 16):
          x = vals1_v[r, pl.ds(16 * k, 16)] * inv1
          vals1_v[r, pl.ds(16 * k, 16)] = jnp.where(x > 0, x, jnp.exp(x) - 1.0)
        return carry2
      lax.fori_loop(0, _OB, row_body, 0)

      obase = pl.multiple_of(node_base + r0, 8)
      pltpu.sync_copy(vals0_v.at[pl.ds(0, _OB), :],
                      out_hbm.at[pl.ds(obase, _OB), pl.ds(0, _D)])
      pltpu.sync_copy(vals1_v.at[pl.ds(0, _OB), :],
                      out_hbm.at[pl.ds(obase, _OB), pl.ds(_D, _D)])
      return carry
    lax.fori_loop(0, nch2, out_body, 0)

    if p == 0:
      lax.fori_loop(0, _OB, zden, 0)
      plsc.subcore_barrier()


_sc_kernel = functools.partial(
    pl.kernel,
    out_type=jax.ShapeDtypeStruct((_N, 2 * _D), jnp.float32),
    mesh=plsc.VectorSubcoreMesh(core_axis_name="c", subcore_axis_name="s"),
    compiler_params=pltpu.CompilerParams(needs_layout_passes=False),
    scratch_types=[
        pltpu.VMEM((2 * _N,), jnp.float32),
        pltpu.VMEM((16,), jnp.int32),
        pltpu.VMEM((_C,), jnp.int32),
        pltpu.VMEM((2 * _C,), jnp.float32),
        pltpu.VMEM((2 * _C,), jnp.float32),
        pltpu.VMEM((_C, _D), jnp.float32),
        pltpu.VMEM((_C,), jnp.float32),
        pltpu.VMEM((_C,), jnp.float32),
        pltpu.VMEM((_C,), jnp.int32),
        pltpu.VMEM((_C, _D), jnp.float32),
        pltpu.VMEM((_C, _D), jnp.float32),
        pltpu.VMEM((_C, _D), jnp.float32),
        pltpu.SemaphoreType.DMA,
        pltpu.SemaphoreType.DMA,
        pltpu.VMEM_SHARED((_NROWS, _D), jnp.float32),
        pltpu.VMEM_SHARED((_NROWS, _D), jnp.float32),
        pltpu.VMEM_SHARED((_NROWS, _D), jnp.float32),
    ],
)(_sc_body)


def _build_rot_weight(r_vec):
  rv = r_vec / jnp.maximum(jnp.linalg.norm(r_vec, axis=2, keepdims=True), 1e-12)
  rv2 = jnp.stack([rv, rv], axis=1)
  rv2 = rv2.at[:, 1, :, 1].set(-rv2[:, 1, :, 1])
  rv2 = rv2.reshape(r_vec.shape[0] * 2, _D // 2, 2)
  final = jnp.zeros((_L, _D // 2, 2), jnp.float32)
  final = final.at[-1, :, 0].set(1.0)
  for i in range(_L - 2, -1, -1):
    re = final[i + 1, :, 0] * rv2[_ETYPES[i], :, 0] - final[i + 1, :, 1] * rv2[_ETYPES[i], :, 1]
    im = final[i + 1, :, 0] * rv2[_ETYPES[i], :, 1] + final[i + 1, :, 1] * rv2[_ETYPES[i], :, 0]
    final = final.at[i, :, 0].set(re)
    final = final.at[i, :, 1].set(im)
  cc = final[:, :, 0]
  ss = final[:, :, 1]
  m = jnp.stack([jnp.stack([cc, ss], -1), jnp.stack([-ss, cc], -1)], -2) / 3.0
  return jnp.einsum("pq,ipab->ipaqb", jnp.eye(_D // 2, dtype=jnp.float32),
                    m).reshape(_L * _D, _D)


def kernel(features, edata, dst_index, attn1_w, attn2, r_vec,
           fusion_w1, fusion_b1, fusion_w2):
  del fusion_w1, fusion_b1, fusion_w2
  w_rot = _build_rot_weight(r_vec).reshape(_L, _D, _D)
  edata_t = jnp.transpose(edata, (1, 0, 2))
  dst2d = dst_index.astype(jnp.int32).reshape(_NEB, _EB)

  a1n, splitv, dstp2d = pl.pallas_call(
      _tc_node_body,
      out_shape=[
          jax.ShapeDtypeStruct((_N, 2), jnp.float32),
          jax.ShapeDtypeStruct((1, 16), jnp.int32),
          jax.ShapeDtypeStruct((_EPAD // _EB, _EB), jnp.int32),
      ],
  )(features, attn1_w, dst2d)

  mp, a2 = pl.pallas_call(
      _tc_edge_body,
      grid=(_NEBP,),
      in_specs=[
          pl.BlockSpec((1, _EB, _D),
                       lambda i: (0, jnp.minimum(i, _NEB - 1), 0)),
          pl.BlockSpec((1, _EB, _D),
                       lambda i: (1, jnp.minimum(i, _NEB - 1), 0)),
          pl.BlockSpec((1, _EB, _D),
                       lambda i: (2, jnp.minimum(i, _NEB - 1), 0)),
          pl.BlockSpec((_L, _D, _D), lambda i: (0, 0, 0)),
          pl.BlockSpec((_H, _D), lambda i: (0, 0)),
      ],
      out_specs=[
          pl.BlockSpec((_EB, _D), lambda i: (i, 0)),
          pl.BlockSpec((2, _EB), lambda i: (0, i)),
      ],
      out_shape=[
          jax.ShapeDtypeStruct((_EPAD, _D), jnp.float32),
          jax.ShapeDtypeStruct((2, _EPAD), jnp.float32),
      ],
  )(edata_t, edata_t, edata_t, w_rot, attn2[0])

  return _sc_kernel(mp, a2, dstp2d.reshape(-1), a1n.reshape(-1),
                    splitv.reshape(-1))
